# baseline (speedup 1.0000x reference)
import functools

import jax
import jax.numpy as jnp
from jax import lax
from jax.experimental import pallas as pl
from jax.experimental.pallas import tpu as pltpu
from jax.experimental.pallas import tpu_sc as plsc

N = 10000
E = 320000
G = 512
D1 = 128
D3 = 256
NC, NS, L = 2, 16, 16
CH = 128

EPAD = 323584
ECH = EPAD // CH
NPAD = 12288
ACC_ROWS = 10240
DEG_ROWS = 10240
CNT_ROWS = 1024
POOL_ROWS = 640

_F32 = jnp.float32


def _mesh():
    return plsc.VectorSubcoreMesh(core_axis_name="c", subcore_axis_name="s")


_SC_PARAMS = pltpu.CompilerParams(use_tc_tiling_on_sc=False)


def _zero_vec(ref, n):
    @pl.loop(0, n, step=L)
    def _(j):
        ref[pl.ds(j, L)] = jnp.zeros((L,), _F32)


def _zero_2d(ref, rows, cols):
    @pl.loop(0, rows)
    def _(r):
        @pl.loop(0, cols, step=L)
        def _(j):
            ref[r, pl.ds(j, L)] = jnp.zeros((L,), _F32)


@functools.partial(
    pl.kernel,
    out_type=(jax.ShapeDtypeStruct((NC, DEG_ROWS), _F32),
              jax.ShapeDtypeStruct((NC, CNT_ROWS), _F32)),
    mesh=_mesh(),
    compiler_params=_SC_PARAMS,
    scratch_types=[
        pltpu.VMEM((79, CH), jnp.int32),
        pltpu.VMEM((3, CH), jnp.int32),
        pltpu.VMEM((CH,), _F32),
        pltpu.VMEM((640,), _F32),
        pltpu.VMEM_SHARED((DEG_ROWS,), _F32),
        pltpu.VMEM_SHARED((CNT_ROWS,), _F32),
    ],
)
def _deg_kernel(dst_hbm, bat_hbm, deg_out, cnt_out, didx, bidx, ones_v, zbuf,
                dacc, cacc):
    c = lax.axis_index("c")
    s = lax.axis_index("s")
    wid = s * NC + c

    @pl.loop(0, CH, step=L)
    def _(j):
        ones_v[pl.ds(j, L)] = jnp.ones((L,), _F32)

    _zero_vec(zbuf, 640)
    pltpu.sync_copy(zbuf, dacc.at[pl.ds(s * 640, 640)])
    pltpu.sync_copy(zbuf.at[pl.ds(0, 64)], cacc.at[pl.ds(s * 64, 64)])
    plsc.subcore_barrier()

    pltpu.sync_copy(dst_hbm.at[pl.ds(wid * 79, 79)], didx)
    pltpu.sync_copy(bat_hbm.at[pl.ds(wid * 3, 3)], bidx)

    @pl.loop(0, 79)
    def _(i):
        pltpu.sync_copy(ones_v, dacc.at[didx.at[i]], add=True)

    @pl.loop(0, 3)
    def _(i):
        pltpu.sync_copy(ones_v, cacc.at[bidx.at[i]], add=True)

    plsc.subcore_barrier()
    pltpu.sync_copy(dacc.at[pl.ds(s * 640, 640)],
                    deg_out.at[c, pl.ds(s * 640, 640)])
    pltpu.sync_copy(cacc.at[pl.ds(s * 64, 64)],
                    cnt_out.at[c, pl.ds(s * 64, 64)])


def _conv_body(tab_hbm, sidx_src, dst_hbm, out_hbm, sidx, didx, rows0, rows1,
               g0, g1, s0, s1, acc, c, s, phases):
    _zero_2d(rows0, CH, D1)

    @pl.loop(0, 5)
    def _(k):
        pltpu.sync_copy(rows0, acc.at[pl.ds(s * 640 + k * CH, CH)])

    plsc.subcore_barrier()

    rows = (rows0, rows1)
    gsems = (g0, g1)
    ssems = (s0, s1)
    for base, nch in phases:
        pltpu.sync_copy(sidx_src.at[pl.ds(base, nch)],
                        sidx.at[pl.ds(0, nch)])
        pltpu.sync_copy(dst_hbm.at[pl.ds(base, nch)],
                        didx.at[pl.ds(0, nch)])
        cpg = [None, None]
        cpg[0] = pltpu.async_copy(tab_hbm.at[sidx.at[0]], rows[0], gsems[0])
        for i in range(nch):
            b = i % 2
            cpg[b].wait()
            if i + 1 < nch:
                nb = (i + 1) % 2
                cpg[nb] = pltpu.async_copy(tab_hbm.at[sidx.at[i + 1]],
                                           rows[nb], gsems[nb])
            pltpu.sync_copy(rows[b], acc.at[didx.at[i]], add=True)

    plsc.subcore_barrier()
    pltpu.sync_copy(acc.at[pl.ds(s * 640, 640)],
                    out_hbm.at[c, pl.ds(s * 640, 640)])


@functools.partial(
    pl.kernel,
    out_type=jax.ShapeDtypeStruct((NC, ACC_ROWS, D1), _F32),
    mesh=_mesh(),
    compiler_params=_SC_PARAMS,
    scratch_types=[
        pltpu.VMEM((40, CH), jnp.int32),
        pltpu.VMEM((40, CH), jnp.int32),
        pltpu.VMEM((CH, D1), _F32),
        pltpu.VMEM((CH, D1), _F32),
        pltpu.SemaphoreType.DMA,
        pltpu.SemaphoreType.DMA,
        pltpu.SemaphoreType.DMA,
        pltpu.SemaphoreType.DMA,
        pltpu.VMEM_SHARED((ACC_ROWS, D1), _F32),
    ],
)
def _conv12_kernel(tab_hbm, src2_hbm, dst_hbm, out_hbm, sidx, didx, rows0,
                   rows1, g0, g1, s0, s1, acc):
    c = lax.axis_index("c")
    s = lax.axis_index("s")
    wid = s * NC + (1 - c)
    _conv_body(tab_hbm, src2_hbm.at[c], dst_hbm, out_hbm, sidx, didx, rows0,
               rows1, g0, g1, s0, s1, acc, c, s,
               [(wid * 79, 40), (wid * 79 + 40, 39)])


@functools.partial(
    pl.kernel,
    out_type=jax.ShapeDtypeStruct((NC, ACC_ROWS, D1), _F32),
    mesh=_mesh(),
    compiler_params=_SC_PARAMS,
    scratch_types=[
        pltpu.VMEM((40, CH), jnp.int32),
        pltpu.VMEM((40, CH), jnp.int32),
        pltpu.VMEM((CH, D1), _F32),
        pltpu.VMEM((CH, D1), _F32),
        pltpu.SemaphoreType.DMA,
        pltpu.SemaphoreType.DMA,
        pltpu.SemaphoreType.DMA,
        pltpu.SemaphoreType.DMA,
        pltpu.VMEM_SHARED((ACC_ROWS, D1), _F32),
    ],
)
def _conv3_kernel(tab_hbm, src2_hbm, dst_hbm, out_hbm, sidx, didx, rows0,
                  rows1, g0, g1, s0, s1, acc):
    c = lax.axis_index("c")
    s = lax.axis_index("s")
    _conv_body(tab_hbm, src2_hbm.at[c], dst_hbm, out_hbm, sidx, didx, rows0,
               rows1, g0, g1, s0, s1, acc, c, s,
               [(s * 158, 40), (s * 158 + 40, 40),
                (s * 158 + 80, 40), (s * 158 + 120, 38)])


@functools.partial(
    pl.kernel,
    out_type=jax.ShapeDtypeStruct((NC, G, D3), _F32),
    mesh=_mesh(),
    compiler_params=_SC_PARAMS,
    scratch_types=[
        pltpu.VMEM((3, CH), jnp.int32),
        pltpu.VMEM((CH, D3), _F32),
        pltpu.VMEM_SHARED((POOL_ROWS, D3), _F32),
    ],
)
def _pool_kernel(h3_hbm, bat_hbm, out_hbm, bidx, rows, acc):
    c = lax.axis_index("c")
    s = lax.axis_index("s")

    _zero_2d(rows, 40, D3)
    pltpu.sync_copy(rows.at[pl.ds(0, 40)], acc.at[pl.ds(s * 40, 40)])
    plsc.subcore_barrier()

    cb = c * 48 + s * 3
    pltpu.sync_copy(bat_hbm.at[pl.ds(cb, 3)], bidx)

    @pl.loop(0, 3)
    def _(i):
        pltpu.sync_copy(h3_hbm.at[pl.ds((cb + i) * CH, CH)], rows)
        pltpu.sync_copy(rows, acc.at[bidx.at[i]], add=True)

    plsc.subcore_barrier()
    pltpu.sync_copy(acc.at[pl.ds(s * 32, 32)],
                    out_hbm.at[c, pl.ds(s * 32, 32)])


def _b1_body(degp, x, w1, hs1_o, dinv_o):
    d = degp[...]
    deg = d[0] + d[1] + 1.0
    dinv = lax.rsqrt(deg)[:N].reshape(N, 1)
    dinv_o[...] = dinv
    hs1 = jnp.dot(x[...], w1[...], preferred_element_type=_F32) * dinv
    hs1_o[0:N, :] = hs1
    hs1_o[N:2 * N, :] = hs1


def _bn_relu(pre, g, be):
    mean = jnp.mean(pre, axis=0)
    var = jnp.mean((pre - mean) ** 2, axis=0)
    h = (pre - mean) * lax.rsqrt(var + 1e-5) * g + be
    return jnp.maximum(h, 0.0)


def _b2_body(aggp, hs1, dinv, b1, g1, be1, w2, o):
    d = dinv[...]
    pre = (aggp[0, :N] + aggp[1, :N] + hs1[0:N, :]) * d + b1[...]
    h = _bn_relu(pre, g1[...], be1[...])
    hs2 = jnp.dot(h, w2[...], preferred_element_type=_F32) * d
    o[0:N, :] = hs2
    o[N:2 * N, :] = hs2


def _b3_body(aggp, hs2, dinv, b2, g2, be2, w3, o):
    d = dinv[...]
    pre = (aggp[0, :N] + aggp[1, :N] + hs2[0:N, :]) * d + b2[...]
    h = _bn_relu(pre, g2[...], be2[...])
    hs3 = jnp.dot(h, w3[...], preferred_element_type=_F32) * d
    o[0:N, :] = hs3[:, 0:D1]
    o[N:2 * N, :] = hs3[:, D1:D3]


def _b4_body(agg3p, hs3s, dinv, b3, o):
    d = dinv[...]
    b = b3[...]
    o[0:N, 0:D1] = (agg3p[0, :N] + hs3s[0:N, :]) * d + b[0:D1]
    o[0:N, D1:D3] = (agg3p[1, :N] + hs3s[N:2 * N, :]) * d + b[D1:D3]
    o[N:NPAD, :] = jnp.zeros((NPAD - N, D3), _F32)


def _b5_body(poolp, cntp, o):
    sums = poolp[0] + poolp[1]
    ct = cntp[0, 0:G] + cntp[1, 0:G]
    o[...] = sums / jnp.maximum(ct, 1.0).reshape(G, 1)


def kernel(x, edge_index, batch, W1, b1, gamma1, beta1, W2, b2, gamma2, beta2,
           W3, b3):
    ei = edge_index.astype(jnp.int32)
    pad_dst = N + (jnp.arange(EPAD - E, dtype=jnp.int32) % (ACC_ROWS - N))
    pad_bat = G + (jnp.arange(NPAD - N, dtype=jnp.int32) % (POOL_ROWS - G))
    src = jnp.concatenate([ei[0], jnp.zeros((EPAD - E,), jnp.int32)])
    dst = jnp.concatenate([ei[1], pad_dst])
    src2 = jnp.stack([src, src + N]).reshape(NC, ECH, CH)
    dst2d = dst.reshape(ECH, CH)
    bat = jnp.concatenate(
        [batch.astype(jnp.int32), pad_bat]).reshape(NPAD // CH, CH)

    degp, cntp = _deg_kernel(dst2d, bat)

    hs1, dinv = pl.pallas_call(
        _b1_body,
        out_shape=(jax.ShapeDtypeStruct((2 * N, D1), _F32),
                   jax.ShapeDtypeStruct((N, 1), _F32)),
    )(degp, x, W1)

    agg1 = _conv12_kernel(hs1, src2, dst2d)

    hs2 = pl.pallas_call(
        _b2_body, out_shape=jax.ShapeDtypeStruct((2 * N, D1), _F32),
    )(agg1, hs1, dinv, b1, gamma1, beta1, W2)

    agg2 = _conv12_kernel(hs2, src2, dst2d)

    hs3s = pl.pallas_call(
        _b3_body, out_shape=jax.ShapeDtypeStruct((2 * N, D1), _F32),
    )(agg2, hs2, dinv, b2, gamma2, beta2, W3)

    agg3 = _conv3_kernel(hs3s, src2, dst2d)

    h3 = pl.pallas_call(
        _b4_body, out_shape=jax.ShapeDtypeStruct((NPAD, D3), _F32),
    )(agg3, hs3s, dinv, b3)

    poolp = _pool_kernel(h3, bat)

    emb = pl.pallas_call(
        _b5_body, out_shape=jax.ShapeDtypeStruct((G, D3), _F32),
    )(poolp, cntp)

    return emb


# speedup vs baseline: 1.8446x; 1.8446x over previous
import functools

import jax
import jax.numpy as jnp
from jax import lax
from jax.experimental import pallas as pl
from jax.experimental.pallas import tpu as pltpu
from jax.experimental.pallas import tpu_sc as plsc

N = 10000
E = 320000
G = 512
D1 = 128
D3 = 256
NC, NS, L = 2, 16, 16
CH = 128

EPAD = 323584
ECH = EPAD // CH
NPAD = 12288
ACC_ROWS = 10240
DEG_ROWS = 10240
CNT_ROWS = 1024
POOL_ROWS = 640

_F32 = jnp.float32


def _mesh():
    return plsc.VectorSubcoreMesh(core_axis_name="c", subcore_axis_name="s")


_SC_PARAMS = pltpu.CompilerParams(use_tc_tiling_on_sc=False)


def _zero_vec(ref, n):
    @pl.loop(0, n, step=L)
    def _(j):
        ref[pl.ds(j, L)] = jnp.zeros((L,), _F32)


def _zero_2d(ref, rows, cols):
    @pl.loop(0, rows)
    def _(r):
        @pl.loop(0, cols, step=L)
        def _(j):
            ref[r, pl.ds(j, L)] = jnp.zeros((L,), _F32)


@functools.partial(
    pl.kernel,
    out_type=(jax.ShapeDtypeStruct((NC, DEG_ROWS), _F32),
              jax.ShapeDtypeStruct((NC, CNT_ROWS), _F32)),
    mesh=_mesh(),
    compiler_params=_SC_PARAMS,
    scratch_types=[
        pltpu.VMEM((79, CH), jnp.int32),
        pltpu.VMEM((3, CH), jnp.int32),
        pltpu.VMEM((CH,), _F32),
        pltpu.VMEM((640,), _F32),
        pltpu.VMEM_SHARED((DEG_ROWS,), _F32),
        pltpu.VMEM_SHARED((CNT_ROWS,), _F32),
    ],
)
def _deg_kernel(dst_hbm, bat_hbm, deg_out, cnt_out, didx, bidx, ones_v, zbuf,
                dacc, cacc):
    c = lax.axis_index("c")
    s = lax.axis_index("s")
    wid = s * NC + c

    @pl.loop(0, CH, step=L)
    def _(j):
        ones_v[pl.ds(j, L)] = jnp.ones((L,), _F32)

    _zero_vec(zbuf, 640)
    pltpu.sync_copy(zbuf, dacc.at[pl.ds(s * 640, 640)])
    pltpu.sync_copy(zbuf.at[pl.ds(0, 64)], cacc.at[pl.ds(s * 64, 64)])
    plsc.subcore_barrier()

    pltpu.sync_copy(dst_hbm.at[pl.ds(wid * 79, 79)], didx)
    pltpu.sync_copy(bat_hbm.at[pl.ds(wid * 3, 3)], bidx)

    @pl.loop(0, 79)
    def _(i):
        pltpu.sync_copy(ones_v, dacc.at[didx.at[i]], add=True)

    @pl.loop(0, 3)
    def _(i):
        pltpu.sync_copy(ones_v, cacc.at[bidx.at[i]], add=True)

    plsc.subcore_barrier()
    pltpu.sync_copy(dacc.at[pl.ds(s * 640, 640)],
                    deg_out.at[c, pl.ds(s * 640, 640)])
    pltpu.sync_copy(cacc.at[pl.ds(s * 64, 64)],
                    cnt_out.at[c, pl.ds(s * 64, 64)])


def _conv_body(tab_hbm, sidx_src, dst_hbm, out_hbm, sidx, didx, rows0, rows1,
               g0, g1, s0, s1, acc, c, s, phases):
    _zero_2d(rows0, CH, D1)

    @pl.loop(0, 5)
    def _(k):
        pltpu.sync_copy(rows0, acc.at[pl.ds(s * 640 + k * CH, CH)])

    plsc.subcore_barrier()

    rows = (rows0, rows1)
    gsems = (g0, g1)
    ssems = (s0, s1)
    for base, nch in phases:
        pltpu.sync_copy(sidx_src.at[pl.ds(base, nch)],
                        sidx.at[pl.ds(0, nch)])
        pltpu.sync_copy(dst_hbm.at[pl.ds(base, nch)],
                        didx.at[pl.ds(0, nch)])
        cpg = [None, None]
        cpg[0] = pltpu.async_copy(tab_hbm.at[sidx.at[0]], rows[0], gsems[0])
        for i in range(nch):
            b = i % 2
            cpg[b].wait()
            if i + 1 < nch:
                nb = (i + 1) % 2
                cpg[nb] = pltpu.async_copy(tab_hbm.at[sidx.at[i + 1]],
                                           rows[nb], gsems[nb])
            pltpu.sync_copy(rows[b], acc.at[didx.at[i]], add=True)

    plsc.subcore_barrier()
    pltpu.sync_copy(acc.at[pl.ds(s * 640, 640)],
                    out_hbm.at[c, pl.ds(s * 640, 640)])


@functools.partial(
    pl.kernel,
    out_type=jax.ShapeDtypeStruct((NC, ACC_ROWS, D1), _F32),
    mesh=_mesh(),
    compiler_params=_SC_PARAMS,
    scratch_types=[
        pltpu.VMEM((40, CH), jnp.int32),
        pltpu.VMEM((40, CH), jnp.int32),
        pltpu.VMEM((CH, D1), _F32),
        pltpu.VMEM((CH, D1), _F32),
        pltpu.SemaphoreType.DMA,
        pltpu.SemaphoreType.DMA,
        pltpu.SemaphoreType.DMA,
        pltpu.SemaphoreType.DMA,
        pltpu.VMEM_SHARED((ACC_ROWS, D1), _F32),
    ],
)
def _conv12_kernel(tab_hbm, src2_hbm, dst_hbm, out_hbm, sidx, didx, rows0,
                   rows1, g0, g1, s0, s1, acc):
    c = lax.axis_index("c")
    s = lax.axis_index("s")
    wid = s * NC + c
    _conv_body(tab_hbm, src2_hbm.at[c], dst_hbm, out_hbm, sidx, didx, rows0,
               rows1, g0, g1, s0, s1, acc, c, s,
               [(wid * 79, 40), (wid * 79 + 40, 39)])


@functools.partial(
    pl.kernel,
    out_type=jax.ShapeDtypeStruct((NC, ACC_ROWS, D1), _F32),
    mesh=_mesh(),
    compiler_params=_SC_PARAMS,
    scratch_types=[
        pltpu.VMEM((40, CH), jnp.int32),
        pltpu.VMEM((40, CH), jnp.int32),
        pltpu.VMEM((CH, D1), _F32),
        pltpu.VMEM((CH, D1), _F32),
        pltpu.SemaphoreType.DMA,
        pltpu.SemaphoreType.DMA,
        pltpu.SemaphoreType.DMA,
        pltpu.SemaphoreType.DMA,
        pltpu.VMEM_SHARED((ACC_ROWS, D1), _F32),
    ],
)
def _conv3_kernel(tab_hbm, src2_hbm, dst_hbm, out_hbm, sidx, didx, rows0,
                  rows1, g0, g1, s0, s1, acc):
    c = lax.axis_index("c")
    s = lax.axis_index("s")
    _conv_body(tab_hbm, src2_hbm.at[c], dst_hbm, out_hbm, sidx, didx, rows0,
               rows1, g0, g1, s0, s1, acc, c, s,
               [(s * 158, 40), (s * 158 + 40, 40),
                (s * 158 + 80, 40), (s * 158 + 120, 38)])


@functools.partial(
    pl.kernel,
    out_type=jax.ShapeDtypeStruct((NC, G, D3), _F32),
    mesh=_mesh(),
    compiler_params=_SC_PARAMS,
    scratch_types=[
        pltpu.VMEM((3, CH), jnp.int32),
        pltpu.VMEM((CH, D3), _F32),
        pltpu.VMEM_SHARED((POOL_ROWS, D3), _F32),
    ],
)
def _pool_kernel(h3_hbm, bat_hbm, out_hbm, bidx, rows, acc):
    c = lax.axis_index("c")
    s = lax.axis_index("s")

    _zero_2d(rows, 40, D3)
    pltpu.sync_copy(rows.at[pl.ds(0, 40)], acc.at[pl.ds(s * 40, 40)])
    plsc.subcore_barrier()

    cb = c * 48 + s * 3
    pltpu.sync_copy(bat_hbm.at[pl.ds(cb, 3)], bidx)

    @pl.loop(0, 3)
    def _(i):
        pltpu.sync_copy(h3_hbm.at[pl.ds((cb + i) * CH, CH)], rows)
        pltpu.sync_copy(rows, acc.at[bidx.at[i]], add=True)

    plsc.subcore_barrier()
    pltpu.sync_copy(acc.at[pl.ds(s * 32, 32)],
                    out_hbm.at[c, pl.ds(s * 32, 32)])


def _b1_body(degp, x, w1, hs1_o, dinv_o):
    d = degp[...]
    deg = d[0] + d[1] + 1.0
    dinv = lax.rsqrt(deg)[:N].reshape(N, 1)
    dinv_o[...] = dinv
    hs1 = jnp.dot(x[...], w1[...], preferred_element_type=_F32) * dinv
    hs1_o[0:N, :] = hs1
    hs1_o[N:2 * N, :] = hs1


def _bn_relu(pre, g, be):
    mean = jnp.mean(pre, axis=0)
    var = jnp.mean((pre - mean) ** 2, axis=0)
    h = (pre - mean) * lax.rsqrt(var + 1e-5) * g + be
    return jnp.maximum(h, 0.0)


def _b2_body(aggp, hs1, dinv, b1, g1, be1, w2, o):
    d = dinv[...]
    pre = (aggp[0, :N] + aggp[1, :N] + hs1[0:N, :]) * d + b1[...]
    h = _bn_relu(pre, g1[...], be1[...])
    hs2 = jnp.dot(h, w2[...], preferred_element_type=_F32) * d
    o[0:N, :] = hs2
    o[N:2 * N, :] = hs2


def _b3_body(aggp, hs2, dinv, b2, g2, be2, w3, o):
    d = dinv[...]
    pre = (aggp[0, :N] + aggp[1, :N] + hs2[0:N, :]) * d + b2[...]
    h = _bn_relu(pre, g2[...], be2[...])
    hs3 = jnp.dot(h, w3[...], preferred_element_type=_F32) * d
    o[0:N, :] = hs3[:, 0:D1]
    o[N:2 * N, :] = hs3[:, D1:D3]


def _b4_body(agg3p, hs3s, dinv, b3, o):
    d = dinv[...]
    b = b3[...]
    o[0:N, 0:D1] = (agg3p[0, :N] + hs3s[0:N, :]) * d + b[0:D1]
    o[0:N, D1:D3] = (agg3p[1, :N] + hs3s[N:2 * N, :]) * d + b[D1:D3]
    o[N:NPAD, :] = jnp.zeros((NPAD - N, D3), _F32)


def _b5_body(poolp, cntp, o):
    sums = poolp[0] + poolp[1]
    ct = cntp[0, 0:G] + cntp[1, 0:G]
    o[...] = sums / jnp.maximum(ct, 1.0).reshape(G, 1)


def kernel(x, edge_index, batch, W1, b1, gamma1, beta1, W2, b2, gamma2, beta2,
           W3, b3):
    ei = edge_index.astype(jnp.int32)
    pad_dst = N + (jnp.arange(EPAD - E, dtype=jnp.int32) % (ACC_ROWS - N))
    pad_bat = G + (jnp.arange(NPAD - N, dtype=jnp.int32) % (POOL_ROWS - G))
    pad_src = jnp.arange(EPAD - E, dtype=jnp.int32) * 7 % N
    src = jnp.concatenate([ei[0], pad_src])
    dst = jnp.concatenate([ei[1], pad_dst])
    src2 = jnp.stack([src, src + N]).reshape(NC, ECH, CH)
    dst2d = dst.reshape(ECH, CH)
    bat = jnp.concatenate(
        [batch.astype(jnp.int32), pad_bat]).reshape(NPAD // CH, CH)

    degp, cntp = _deg_kernel(dst2d, bat)

    hs1, dinv = pl.pallas_call(
        _b1_body,
        out_shape=(jax.ShapeDtypeStruct((2 * N, D1), _F32),
                   jax.ShapeDtypeStruct((N, 1), _F32)),
    )(degp, x, W1)

    agg1 = _conv12_kernel(hs1, src2, dst2d)

    hs2 = pl.pallas_call(
        _b2_body, out_shape=jax.ShapeDtypeStruct((2 * N, D1), _F32),
    )(agg1, hs1, dinv, b1, gamma1, beta1, W2)

    agg2 = _conv12_kernel(hs2, src2, dst2d)

    hs3s = pl.pallas_call(
        _b3_body, out_shape=jax.ShapeDtypeStruct((2 * N, D1), _F32),
    )(agg2, hs2, dinv, b2, gamma2, beta2, W3)

    agg3 = _conv3_kernel(hs3s, src2, dst2d)

    h3 = pl.pallas_call(
        _b4_body, out_shape=jax.ShapeDtypeStruct((NPAD, D3), _F32),
    )(agg3, hs3s, dinv, b3)

    poolp = _pool_kernel(h3, bat)

    emb = pl.pallas_call(
        _b5_body, out_shape=jax.ShapeDtypeStruct((G, D3), _F32),
    )(poolp, cntp)

    return emb


# speedup vs baseline: 1.8498x; 1.0028x over previous
import functools

import jax
import jax.numpy as jnp
from jax import lax
from jax.experimental import pallas as pl
from jax.experimental.pallas import tpu as pltpu
from jax.experimental.pallas import tpu_sc as plsc

N = 10000
E = 320000
G = 512
D1 = 128
D3 = 256
NC, NS, L = 2, 16, 16
CH = 128

EPAD = 323584
ECH = EPAD // CH
NPAD = 12288
ACC_ROWS = 10240
DEG_ROWS = 10240
CNT_ROWS = 1024
POOL_ROWS = 640

_F32 = jnp.float32


def _mesh():
    return plsc.VectorSubcoreMesh(core_axis_name="c", subcore_axis_name="s")


_SC_PARAMS = pltpu.CompilerParams(use_tc_tiling_on_sc=False)


def _zero_vec(ref, n):
    @pl.loop(0, n, step=L)
    def _(j):
        ref[pl.ds(j, L)] = jnp.zeros((L,), _F32)


def _zero_2d(ref, rows, cols):
    @pl.loop(0, rows)
    def _(r):
        @pl.loop(0, cols, step=L)
        def _(j):
            ref[r, pl.ds(j, L)] = jnp.zeros((L,), _F32)


@functools.partial(
    pl.kernel,
    out_type=(jax.ShapeDtypeStruct((NC, DEG_ROWS), _F32),
              jax.ShapeDtypeStruct((NC, CNT_ROWS), _F32)),
    mesh=_mesh(),
    compiler_params=_SC_PARAMS,
    scratch_types=[
        pltpu.VMEM((79, CH), jnp.int32),
        pltpu.VMEM((3, CH), jnp.int32),
        pltpu.VMEM((CH,), _F32),
        pltpu.VMEM((640,), _F32),
        pltpu.VMEM_SHARED((DEG_ROWS,), _F32),
        pltpu.VMEM_SHARED((CNT_ROWS,), _F32),
    ],
)
def _deg_kernel(dst_hbm, bat_hbm, deg_out, cnt_out, didx, bidx, ones_v, zbuf,
                dacc, cacc):
    c = lax.axis_index("c")
    s = lax.axis_index("s")
    wid = s * NC + c

    @pl.loop(0, CH, step=L)
    def _(j):
        ones_v[pl.ds(j, L)] = jnp.ones((L,), _F32)

    _zero_vec(zbuf, 640)
    pltpu.sync_copy(zbuf, dacc.at[pl.ds(s * 640, 640)])
    pltpu.sync_copy(zbuf.at[pl.ds(0, 64)], cacc.at[pl.ds(s * 64, 64)])
    plsc.subcore_barrier()

    pltpu.sync_copy(dst_hbm.at[pl.ds(wid * 79, 79)], didx)
    pltpu.sync_copy(bat_hbm.at[pl.ds(wid * 3, 3)], bidx)

    @pl.loop(0, 79)
    def _(i):
        pltpu.sync_copy(ones_v, dacc.at[didx.at[i]], add=True)

    @pl.loop(0, 3)
    def _(i):
        pltpu.sync_copy(ones_v, cacc.at[bidx.at[i]], add=True)

    plsc.subcore_barrier()
    pltpu.sync_copy(dacc.at[pl.ds(s * 640, 640)],
                    deg_out.at[c, pl.ds(s * 640, 640)])
    pltpu.sync_copy(cacc.at[pl.ds(s * 64, 64)],
                    cnt_out.at[c, pl.ds(s * 64, 64)])


def _conv_body(tab_hbm, sidx_src, dst_hbm, out_hbm, sidx, didx, rows0, rows1,
               g0, g1, acc, c, s, phases):
    _zero_2d(rows0, CH, D1)

    @pl.loop(0, 5)
    def _(k):
        pltpu.sync_copy(rows0, acc.at[pl.ds(s * 640 + k * CH, CH)])

    plsc.subcore_barrier()

    rows = (rows0, rows1)
    gsems = (g0, g1)
    for base, nch in phases:
        pltpu.sync_copy(sidx_src.at[pl.ds(base, nch)],
                        sidx.at[pl.ds(0, nch)])
        pltpu.sync_copy(dst_hbm.at[pl.ds(base, nch)],
                        didx.at[pl.ds(0, nch)])
        cpg = [None, None]
        cpg[0] = pltpu.async_copy(tab_hbm.at[sidx.at[0]], rows[0], gsems[0])
        for i in range(nch):
            b = i % 2
            cpg[b].wait()
            if i + 1 < nch:
                nb = (i + 1) % 2
                cpg[nb] = pltpu.async_copy(tab_hbm.at[sidx.at[i + 1]],
                                           rows[nb], gsems[nb])
            pltpu.sync_copy(rows[b], acc.at[didx.at[i]], add=True)

    plsc.subcore_barrier()
    pltpu.sync_copy(acc.at[pl.ds(s * 640, 640)],
                    out_hbm.at[c, pl.ds(s * 640, 640)])


@functools.partial(
    pl.kernel,
    out_type=jax.ShapeDtypeStruct((NC, ACC_ROWS, D1), _F32),
    mesh=_mesh(),
    compiler_params=_SC_PARAMS,
    scratch_types=[
        pltpu.VMEM((40, CH), jnp.int32),
        pltpu.VMEM((40, CH), jnp.int32),
        pltpu.VMEM((CH, D1), _F32),
        pltpu.VMEM((CH, D1), _F32),
        pltpu.SemaphoreType.DMA,
        pltpu.SemaphoreType.DMA,
        pltpu.VMEM_SHARED((ACC_ROWS, D1), _F32),
    ],
)
def _conv12_kernel(tab_hbm, src2_hbm, dst_hbm, out_hbm, sidx, didx, rows0,
                   rows1, g0, g1, acc):
    c = lax.axis_index("c")
    s = lax.axis_index("s")
    wid = s * NC + c
    _conv_body(tab_hbm, src2_hbm.at[c], dst_hbm, out_hbm, sidx, didx, rows0,
               rows1, g0, g1, acc, c, s,
               [(wid * 79, 40), (wid * 79 + 40, 39)])


@functools.partial(
    pl.kernel,
    out_type=jax.ShapeDtypeStruct((NC, ACC_ROWS, D1), _F32),
    mesh=_mesh(),
    compiler_params=_SC_PARAMS,
    scratch_types=[
        pltpu.VMEM((40, CH), jnp.int32),
        pltpu.VMEM((40, CH), jnp.int32),
        pltpu.VMEM((CH, D1), _F32),
        pltpu.VMEM((CH, D1), _F32),
        pltpu.SemaphoreType.DMA,
        pltpu.SemaphoreType.DMA,
        pltpu.VMEM_SHARED((ACC_ROWS, D1), _F32),
    ],
)
def _conv3_kernel(tab_hbm, src2_hbm, dst_hbm, out_hbm, sidx, didx, rows0,
                  rows1, g0, g1, acc):
    c = lax.axis_index("c")
    s = lax.axis_index("s")
    _conv_body(tab_hbm, src2_hbm.at[c], dst_hbm, out_hbm, sidx, didx, rows0,
               rows1, g0, g1, acc, c, s,
               [(s * 158, 40), (s * 158 + 40, 40),
                (s * 158 + 80, 40), (s * 158 + 120, 38)])


@functools.partial(
    pl.kernel,
    out_type=jax.ShapeDtypeStruct((NC, G, D3), _F32),
    mesh=_mesh(),
    compiler_params=_SC_PARAMS,
    scratch_types=[
        pltpu.VMEM((3, CH), jnp.int32),
        pltpu.VMEM((CH, D3), _F32),
        pltpu.VMEM_SHARED((POOL_ROWS, D3), _F32),
    ],
)
def _pool_kernel(h3_hbm, bat_hbm, out_hbm, bidx, rows, acc):
    c = lax.axis_index("c")
    s = lax.axis_index("s")

    _zero_2d(rows, 40, D3)
    pltpu.sync_copy(rows.at[pl.ds(0, 40)], acc.at[pl.ds(s * 40, 40)])
    plsc.subcore_barrier()

    cb = c * 48 + s * 3
    pltpu.sync_copy(bat_hbm.at[pl.ds(cb, 3)], bidx)

    @pl.loop(0, 3)
    def _(i):
        pltpu.sync_copy(h3_hbm.at[pl.ds((cb + i) * CH, CH)], rows)
        pltpu.sync_copy(rows, acc.at[bidx.at[i]], add=True)

    plsc.subcore_barrier()
    pltpu.sync_copy(acc.at[pl.ds(s * 32, 32)],
                    out_hbm.at[c, pl.ds(s * 32, 32)])


def _b1_body(degp, x, w1, hs1_o, dinv_o):
    d = degp[...]
    deg = d[0] + d[1] + 1.0
    dinv = lax.rsqrt(deg)[:N].reshape(N, 1)
    dinv_o[...] = dinv
    hs1 = jnp.dot(x[...], w1[...], preferred_element_type=_F32) * dinv
    hs1_o[0:N, :] = hs1
    hs1_o[N:2 * N, :] = hs1


def _bn_relu(pre, g, be):
    mean = jnp.mean(pre, axis=0)
    var = jnp.mean((pre - mean) ** 2, axis=0)
    h = (pre - mean) * lax.rsqrt(var + 1e-5) * g + be
    return jnp.maximum(h, 0.0)


def _b2_body(aggp, hs1, dinv, b1, g1, be1, w2, o):
    d = dinv[...]
    pre = (aggp[0, :N] + aggp[1, :N] + hs1[0:N, :]) * d + b1[...]
    h = _bn_relu(pre, g1[...], be1[...])
    hs2 = jnp.dot(h, w2[...], preferred_element_type=_F32) * d
    o[0:N, :] = hs2
    o[N:2 * N, :] = hs2


def _b3_body(aggp, hs2, dinv, b2, g2, be2, w3, o):
    d = dinv[...]
    pre = (aggp[0, :N] + aggp[1, :N] + hs2[0:N, :]) * d + b2[...]
    h = _bn_relu(pre, g2[...], be2[...])
    hs3 = jnp.dot(h, w3[...], preferred_element_type=_F32) * d
    o[0:N, :] = hs3[:, 0:D1]
    o[N:2 * N, :] = hs3[:, D1:D3]


def _b4_body(agg3p, hs3s, dinv, b3, o):
    d = dinv[...]
    b = b3[...]
    o[0:N, 0:D1] = (agg3p[0, :N] + hs3s[0:N, :]) * d + b[0:D1]
    o[0:N, D1:D3] = (agg3p[1, :N] + hs3s[N:2 * N, :]) * d + b[D1:D3]
    o[N:NPAD, :] = jnp.zeros((NPAD - N, D3), _F32)


def _b5_body(poolp, cntp, o):
    sums = poolp[0] + poolp[1]
    ct = cntp[0, 0:G] + cntp[1, 0:G]
    o[...] = sums / jnp.maximum(ct, 1.0).reshape(G, 1)


def kernel(x, edge_index, batch, W1, b1, gamma1, beta1, W2, b2, gamma2, beta2,
           W3, b3):
    ei = edge_index.astype(jnp.int32)
    pad_dst = N + (jnp.arange(EPAD - E, dtype=jnp.int32) % (ACC_ROWS - N))
    pad_bat = G + (jnp.arange(NPAD - N, dtype=jnp.int32) % (POOL_ROWS - G))
    pad_src = jnp.arange(EPAD - E, dtype=jnp.int32) * 7 % N
    src = jnp.concatenate([ei[0], pad_src])
    dst = jnp.concatenate([ei[1], pad_dst])
    src2 = jnp.stack([src, src + N]).reshape(NC, ECH, CH)
    dst2d = dst.reshape(ECH, CH)
    bat = jnp.concatenate(
        [batch.astype(jnp.int32), pad_bat]).reshape(NPAD // CH, CH)

    degp, cntp = _deg_kernel(dst2d, bat)

    hs1, dinv = pl.pallas_call(
        _b1_body,
        out_shape=(jax.ShapeDtypeStruct((2 * N, D1), _F32),
                   jax.ShapeDtypeStruct((N, 1), _F32)),
    )(degp, x, W1)

    agg1 = _conv12_kernel(hs1, src2, dst2d)

    hs2 = pl.pallas_call(
        _b2_body, out_shape=jax.ShapeDtypeStruct((2 * N, D1), _F32),
    )(agg1, hs1, dinv, b1, gamma1, beta1, W2)

    agg2 = _conv12_kernel(hs2, src2, dst2d)

    hs3s = pl.pallas_call(
        _b3_body, out_shape=jax.ShapeDtypeStruct((2 * N, D1), _F32),
    )(agg2, hs2, dinv, b2, gamma2, beta2, W3)

    agg3 = _conv3_kernel(hs3s, src2, dst2d)

    h3 = pl.pallas_call(
        _b4_body, out_shape=jax.ShapeDtypeStruct((NPAD, D3), _F32),
    )(agg3, hs3s, dinv, b3)

    poolp = _pool_kernel(h3, bat)

    emb = pl.pallas_call(
        _b5_body, out_shape=jax.ShapeDtypeStruct((G, D3), _F32),
    )(poolp, cntp)

    return emb
